# baseline (speedup 1.0000x reference)
import functools

import jax
import jax.numpy as jnp
from jax import lax
from jax.experimental import pallas as pl
from jax.experimental.pallas import tpu as pltpu
from jax.experimental.pallas import tpu_sc as plsc

M = 1000000
B = 16384
D = 32
L = 16

R = 2048
RN = 2048

NC = 2
NS = 16
NW = NC * NS
PERW = B // NW
CH = 128
NCH = PERW // CH
VL = 16

LPADBIG = 48584


def _copy_body(bel_ref, prob_ref, fout_ref):
    fout_ref[:, 0:D] = bel_ref[...]
    fout_ref[:, D:D + 1] = prob_ref[...].reshape(R, 1)


_copy_call = pl.pallas_call(
    _copy_body,
    grid=(pl.cdiv(M, R),),
    in_specs=[
        pl.BlockSpec((R, D), lambda i: (i, 0)),
        pl.BlockSpec((R,), lambda i: (i,)),
    ],
    out_specs=pl.BlockSpec((R, D + 1), lambda i: (i, 0)),
    out_shape=jax.ShapeDtypeStruct((M, D + 1), jnp.float32),
)


def _node_body(nbel_ref, nprob_ref, n33_ref):
    n33_ref[:, 0:D] = nbel_ref[...]
    n33_ref[:, D:D + 1] = nprob_ref[...].reshape(RN, 1)


_node_call = pl.pallas_call(
    _node_body,
    grid=(B // RN,),
    in_specs=[
        pl.BlockSpec((RN, D), lambda i: (i, 0)),
        pl.BlockSpec((RN,), lambda i: (i,)),
    ],
    out_specs=pl.BlockSpec((RN, D + 1), lambda i: (i, 0)),
    out_shape=jax.ShapeDtypeStruct((B, D + 1), jnp.float32),
)


_sc_mesh = plsc.VectorSubcoreMesh(core_axis_name="c", subcore_axis_name="s")


GN = 32
NG = PERW // GN


@functools.partial(
    pl.kernel,
    mesh=_sc_mesh,
    out_type=(),
    scratch_types=[
        pltpu.VMEM((NCH, CH), jnp.int32),
        pltpu.VMEM((1, PERW), jnp.int32),
        pltpu.SemaphoreType.DMA,
    ],
)
def _sc_lens(l_ref, nlen_hbm, idx2_hbm, idx_v, l_v, lsem):
    cid = lax.axis_index("c")
    sid = lax.axis_index("s")
    wid = sid * NC + cid
    pltpu.sync_copy(idx2_hbm.at[pl.ds(wid * NCH, NCH)], idx_v)
    pltpu.sync_copy(nlen_hbm.at[pl.ds(wid, 1)], l_v)
    for k in range(NCH):
        pltpu.async_copy(l_v.at[0, pl.ds(k * CH, CH)],
                         l_ref.at[idx_v.at[k]], lsem).wait()


@functools.partial(
    pl.kernel,
    mesh=_sc_mesh,
    out_type=(),
    scratch_types=[
        pltpu.VMEM((1, PERW), jnp.int32),
        pltpu.SemaphoreType.DMA,
    ],
)
def _sc_scatter_rows(dst_ref, src_hbm, idxw_hbm, idx_v, sem):
    cid = lax.axis_index("c")
    sid = lax.axis_index("s")
    wid = sid * NC + cid
    base = wid * PERW
    pltpu.sync_copy(idxw_hbm.at[pl.ds(wid, 1)], idx_v)

    def fire(g):
        copies = []
        for h in range(GN // VL):
            off = g * GN + h * VL
            ivec = idx_v[0, pl.ds(off, VL)]
            for t in range(VL):
                i = base + off + t
                r = ivec[t]
                copies.append(pltpu.make_async_copy(
                    src_hbm.at[pl.ds(i, 1)], dst_ref.at[pl.ds(r, 1)], sem))
        for cp in copies:
            cp.start()

    def drain_one():
        for _ in range(GN):
            pltpu.make_async_copy(
                src_hbm.at[pl.ds(0, 1)], dst_ref.at[pl.ds(0, 1)], sem).wait()

    PIPE = 2
    for g in range(PIPE):
        fire(g)

    def step(g, carry):
        fire(g)
        drain_one()
        return carry

    lax.fori_loop(PIPE, NG, step, 0)
    for _ in range(PIPE):
        drain_one()


def kernel(belief_states_mem, probabilities_mem, sequences_mem,
           sequence_lengths_mem, node_belief_states, node_probabilities,
           node_sequences, node_sequence_lengths, idx):
    idx2 = idx.reshape(B // CH, CH)
    idxw = idx.reshape(NW, PERW)
    nlen2 = node_sequence_lengths.reshape(NW, PERW)
    pad = jnp.zeros((LPADBIG,), jnp.int32)
    l_r = jax.new_ref(jnp.concatenate([sequence_lengths_mem, pad]))
    _sc_lens(l_r, nlen2, idx2)
    s_r = jax.new_ref(sequences_mem)
    _sc_scatter_rows(s_r, node_sequences, idxw)
    fout = _copy_call(belief_states_mem, probabilities_mem)
    n33 = _node_call(node_belief_states, node_probabilities)
    f_r = jax.new_ref(fout)
    _sc_scatter_rows(f_r, n33, idxw)
    return (f_r[...], s_r[...], l_r[...][:M], jnp.asarray(B, jnp.int32))


# speedup vs baseline: 1.1423x; 1.1423x over previous
import functools

import jax
import jax.numpy as jnp
from jax import lax
from jax.experimental import pallas as pl
from jax.experimental.pallas import tpu as pltpu
from jax.experimental.pallas import tpu_sc as plsc

M = 1000000
B = 16384
D = 32
L = 16

R = 8192
RN = 2048

NC = 2
NS = 16
NW = NC * NS
PERW = B // NW
CH = 128
NCH = PERW // CH
VL = 16

LPADBIG = 48584


def _copy_body(bel_ref, prob_ref, fout_ref):
    fout_ref[:, 0:D] = bel_ref[...]
    fout_ref[:, D:D + 1] = prob_ref[...].reshape(R, 1)


_copy_call = pl.pallas_call(
    _copy_body,
    grid=(pl.cdiv(M, R),),
    in_specs=[
        pl.BlockSpec((R, D), lambda i: (i, 0)),
        pl.BlockSpec((R,), lambda i: (i,)),
    ],
    out_specs=pl.BlockSpec((R, D + 1), lambda i: (i, 0)),
    out_shape=jax.ShapeDtypeStruct((M, D + 1), jnp.float32),
)


def _node_body(nbel_ref, nprob_ref, n33_ref):
    n33_ref[:, 0:D] = nbel_ref[...]
    n33_ref[:, D:D + 1] = nprob_ref[...].reshape(RN, 1)


_node_call = pl.pallas_call(
    _node_body,
    grid=(B // RN,),
    in_specs=[
        pl.BlockSpec((RN, D), lambda i: (i, 0)),
        pl.BlockSpec((RN,), lambda i: (i,)),
    ],
    out_specs=pl.BlockSpec((RN, D + 1), lambda i: (i, 0)),
    out_shape=jax.ShapeDtypeStruct((B, D + 1), jnp.float32),
)


_sc_mesh = plsc.VectorSubcoreMesh(core_axis_name="c", subcore_axis_name="s")


GN = 32
NG = PERW // GN


@functools.partial(
    pl.kernel,
    mesh=_sc_mesh,
    out_type=(),
    scratch_types=[
        pltpu.VMEM((NCH, CH), jnp.int32),
        pltpu.VMEM((1, PERW), jnp.int32),
        pltpu.SemaphoreType.DMA,
    ],
)
def _sc_lens(l_ref, nlen_hbm, idx2_hbm, idx_v, l_v, lsem):
    cid = lax.axis_index("c")
    sid = lax.axis_index("s")
    wid = sid * NC + cid
    pltpu.sync_copy(idx2_hbm.at[pl.ds(wid * NCH, NCH)], idx_v)
    pltpu.sync_copy(nlen_hbm.at[pl.ds(wid, 1)], l_v)
    for k in range(NCH):
        pltpu.async_copy(l_v.at[0, pl.ds(k * CH, CH)],
                         l_ref.at[idx_v.at[k]], lsem).wait()


@functools.partial(
    pl.kernel,
    mesh=_sc_mesh,
    out_type=(),
    scratch_types=[
        pltpu.VMEM((1, PERW), jnp.int32),
        pltpu.SemaphoreType.DMA,
    ],
)
def _sc_scatter_rows(dst_ref, src_hbm, idxw_hbm, idx_v, sem):
    cid = lax.axis_index("c")
    sid = lax.axis_index("s")
    wid = sid * NC + cid
    base = wid * PERW
    pltpu.sync_copy(idxw_hbm.at[pl.ds(wid, 1)], idx_v)

    def fire(g):
        copies = []
        for h in range(GN // VL):
            off = g * GN + h * VL
            ivec = idx_v[0, pl.ds(off, VL)]
            for t in range(VL):
                i = base + off + t
                r = ivec[t]
                copies.append(pltpu.make_async_copy(
                    src_hbm.at[pl.ds(i, 1)], dst_ref.at[pl.ds(r, 1)], sem))
        for cp in copies:
            cp.start()

    def drain_one():
        for _ in range(GN):
            pltpu.make_async_copy(
                src_hbm.at[pl.ds(0, 1)], dst_ref.at[pl.ds(0, 1)], sem).wait()

    PIPE = 2
    for g in range(PIPE):
        fire(g)

    def step(g, carry):
        fire(g)
        drain_one()
        return carry

    lax.fori_loop(PIPE, NG, step, 0)
    for _ in range(PIPE):
        drain_one()


def kernel(belief_states_mem, probabilities_mem, sequences_mem,
           sequence_lengths_mem, node_belief_states, node_probabilities,
           node_sequences, node_sequence_lengths, idx):
    idx2 = idx.reshape(B // CH, CH)
    idxw = idx.reshape(NW, PERW)
    nlen2 = node_sequence_lengths.reshape(NW, PERW)
    pad = jnp.zeros((LPADBIG,), jnp.int32)
    l_r = jax.new_ref(jnp.concatenate([sequence_lengths_mem, pad]))
    _sc_lens(l_r, nlen2, idx2)
    s_r = jax.new_ref(sequences_mem)
    _sc_scatter_rows(s_r, node_sequences, idxw)
    fout = _copy_call(belief_states_mem, probabilities_mem)
    n33 = _node_call(node_belief_states, node_probabilities)
    f_r = jax.new_ref(fout)
    _sc_scatter_rows(f_r, n33, idxw)
    return (f_r[...], s_r[...], l_r[...][:M], jnp.asarray(B, jnp.int32))


# speedup vs baseline: 1.1467x; 1.0039x over previous
import functools

import jax
import jax.numpy as jnp
from jax import lax
from jax.experimental import pallas as pl
from jax.experimental.pallas import tpu as pltpu
from jax.experimental.pallas import tpu_sc as plsc

M = 1000000
B = 16384
D = 32
L = 16

R = 16384
RN = 2048

NC = 2
NS = 16
NW = NC * NS
PERW = B // NW
CH = 128
NCH = PERW // CH
VL = 16

LPADBIG = 48584


def _copy_body(bel_ref, prob_ref, fout_ref):
    fout_ref[:, 0:D] = bel_ref[...]
    fout_ref[:, D:D + 1] = prob_ref[...].reshape(R, 1)


_copy_call = pl.pallas_call(
    _copy_body,
    grid=(pl.cdiv(M, R),),
    in_specs=[
        pl.BlockSpec((R, D), lambda i: (i, 0)),
        pl.BlockSpec((R,), lambda i: (i,)),
    ],
    out_specs=pl.BlockSpec((R, D + 1), lambda i: (i, 0)),
    out_shape=jax.ShapeDtypeStruct((M, D + 1), jnp.float32),
)


def _node_body(nbel_ref, nprob_ref, n33_ref):
    n33_ref[:, 0:D] = nbel_ref[...]
    n33_ref[:, D:D + 1] = nprob_ref[...].reshape(RN, 1)


_node_call = pl.pallas_call(
    _node_body,
    grid=(B // RN,),
    in_specs=[
        pl.BlockSpec((RN, D), lambda i: (i, 0)),
        pl.BlockSpec((RN,), lambda i: (i,)),
    ],
    out_specs=pl.BlockSpec((RN, D + 1), lambda i: (i, 0)),
    out_shape=jax.ShapeDtypeStruct((B, D + 1), jnp.float32),
)


_sc_mesh = plsc.VectorSubcoreMesh(core_axis_name="c", subcore_axis_name="s")


GN = 32
NG = PERW // GN


@functools.partial(
    pl.kernel,
    mesh=_sc_mesh,
    out_type=(),
    scratch_types=[
        pltpu.VMEM((NCH, CH), jnp.int32),
        pltpu.VMEM((1, PERW), jnp.int32),
        pltpu.SemaphoreType.DMA,
    ],
)
def _sc_lens(l_ref, nlen_hbm, idx2_hbm, idx_v, l_v, lsem):
    cid = lax.axis_index("c")
    sid = lax.axis_index("s")
    wid = sid * NC + cid
    pltpu.sync_copy(idx2_hbm.at[pl.ds(wid * NCH, NCH)], idx_v)
    pltpu.sync_copy(nlen_hbm.at[pl.ds(wid, 1)], l_v)
    for k in range(NCH):
        pltpu.async_copy(l_v.at[0, pl.ds(k * CH, CH)],
                         l_ref.at[idx_v.at[k]], lsem).wait()


@functools.partial(
    pl.kernel,
    mesh=_sc_mesh,
    out_type=(),
    scratch_types=[
        pltpu.VMEM((1, PERW), jnp.int32),
        pltpu.SemaphoreType.DMA,
    ],
)
def _sc_scatter_rows(dst_ref, src_hbm, idxw_hbm, idx_v, sem):
    cid = lax.axis_index("c")
    sid = lax.axis_index("s")
    wid = sid * NC + cid
    base = wid * PERW
    pltpu.sync_copy(idxw_hbm.at[pl.ds(wid, 1)], idx_v)

    def fire(g):
        copies = []
        for h in range(GN // VL):
            off = g * GN + h * VL
            ivec = idx_v[0, pl.ds(off, VL)]
            for t in range(VL):
                i = base + off + t
                r = ivec[t]
                copies.append(pltpu.make_async_copy(
                    src_hbm.at[pl.ds(i, 1)], dst_ref.at[pl.ds(r, 1)], sem))
        for cp in copies:
            cp.start()

    def drain_one():
        for _ in range(GN):
            pltpu.make_async_copy(
                src_hbm.at[pl.ds(0, 1)], dst_ref.at[pl.ds(0, 1)], sem).wait()

    PIPE = 2
    for g in range(PIPE):
        fire(g)

    def step(g, carry):
        fire(g)
        drain_one()
        return carry

    lax.fori_loop(PIPE, NG, step, 0)
    for _ in range(PIPE):
        drain_one()


def kernel(belief_states_mem, probabilities_mem, sequences_mem,
           sequence_lengths_mem, node_belief_states, node_probabilities,
           node_sequences, node_sequence_lengths, idx):
    idx2 = idx.reshape(B // CH, CH)
    idxw = idx.reshape(NW, PERW)
    nlen2 = node_sequence_lengths.reshape(NW, PERW)
    pad = jnp.zeros((LPADBIG,), jnp.int32)
    l_r = jax.new_ref(jnp.concatenate([sequence_lengths_mem, pad]))
    _sc_lens(l_r, nlen2, idx2)
    s_r = jax.new_ref(sequences_mem)
    _sc_scatter_rows(s_r, node_sequences, idxw)
    fout = _copy_call(belief_states_mem, probabilities_mem)
    n33 = _node_call(node_belief_states, node_probabilities)
    f_r = jax.new_ref(fout)
    _sc_scatter_rows(f_r, n33, idxw)
    return (f_r[...], s_r[...], l_r[...][:M], jnp.asarray(B, jnp.int32))


# speedup vs baseline: 1.1475x; 1.0007x over previous
import functools

import jax
import jax.numpy as jnp
from jax import lax
from jax.experimental import pallas as pl
from jax.experimental.pallas import tpu as pltpu
from jax.experimental.pallas import tpu_sc as plsc

M = 1000000
B = 16384
D = 32
L = 16

R = 24576
RN = 2048

NC = 2
NS = 16
NW = NC * NS
PERW = B // NW
CH = 128
NCH = PERW // CH
VL = 16

LPADBIG = 48584


def _copy_body(bel_ref, prob_ref, fout_ref):
    fout_ref[:, 0:D] = bel_ref[...]
    fout_ref[:, D:D + 1] = prob_ref[...].reshape(R, 1)


_copy_call = pl.pallas_call(
    _copy_body,
    grid=(pl.cdiv(M, R),),
    in_specs=[
        pl.BlockSpec((R, D), lambda i: (i, 0)),
        pl.BlockSpec((R,), lambda i: (i,)),
    ],
    out_specs=pl.BlockSpec((R, D + 1), lambda i: (i, 0)),
    out_shape=jax.ShapeDtypeStruct((M, D + 1), jnp.float32),
)


def _node_body(nbel_ref, nprob_ref, n33_ref):
    n33_ref[:, 0:D] = nbel_ref[...]
    n33_ref[:, D:D + 1] = nprob_ref[...].reshape(RN, 1)


_node_call = pl.pallas_call(
    _node_body,
    grid=(B // RN,),
    in_specs=[
        pl.BlockSpec((RN, D), lambda i: (i, 0)),
        pl.BlockSpec((RN,), lambda i: (i,)),
    ],
    out_specs=pl.BlockSpec((RN, D + 1), lambda i: (i, 0)),
    out_shape=jax.ShapeDtypeStruct((B, D + 1), jnp.float32),
)


_sc_mesh = plsc.VectorSubcoreMesh(core_axis_name="c", subcore_axis_name="s")


GN = 32
NG = PERW // GN


@functools.partial(
    pl.kernel,
    mesh=_sc_mesh,
    out_type=(),
    scratch_types=[
        pltpu.VMEM((NCH, CH), jnp.int32),
        pltpu.VMEM((1, PERW), jnp.int32),
        pltpu.SemaphoreType.DMA,
    ],
)
def _sc_lens(l_ref, nlen_hbm, idx2_hbm, idx_v, l_v, lsem):
    cid = lax.axis_index("c")
    sid = lax.axis_index("s")
    wid = sid * NC + cid
    pltpu.sync_copy(idx2_hbm.at[pl.ds(wid * NCH, NCH)], idx_v)
    pltpu.sync_copy(nlen_hbm.at[pl.ds(wid, 1)], l_v)
    for k in range(NCH):
        pltpu.async_copy(l_v.at[0, pl.ds(k * CH, CH)],
                         l_ref.at[idx_v.at[k]], lsem).wait()


@functools.partial(
    pl.kernel,
    mesh=_sc_mesh,
    out_type=(),
    scratch_types=[
        pltpu.VMEM((1, PERW), jnp.int32),
        pltpu.SemaphoreType.DMA,
    ],
)
def _sc_scatter_rows(dst_ref, src_hbm, idxw_hbm, idx_v, sem):
    cid = lax.axis_index("c")
    sid = lax.axis_index("s")
    wid = sid * NC + cid
    base = wid * PERW
    pltpu.sync_copy(idxw_hbm.at[pl.ds(wid, 1)], idx_v)

    def fire(g):
        copies = []
        for h in range(GN // VL):
            off = g * GN + h * VL
            ivec = idx_v[0, pl.ds(off, VL)]
            for t in range(VL):
                i = base + off + t
                r = ivec[t]
                copies.append(pltpu.make_async_copy(
                    src_hbm.at[pl.ds(i, 1)], dst_ref.at[pl.ds(r, 1)], sem))
        for cp in copies:
            cp.start()

    def drain_one():
        for _ in range(GN):
            pltpu.make_async_copy(
                src_hbm.at[pl.ds(0, 1)], dst_ref.at[pl.ds(0, 1)], sem).wait()

    PIPE = 2
    for g in range(PIPE):
        fire(g)

    def step(g, carry):
        fire(g)
        drain_one()
        return carry

    lax.fori_loop(PIPE, NG, step, 0)
    for _ in range(PIPE):
        drain_one()


def kernel(belief_states_mem, probabilities_mem, sequences_mem,
           sequence_lengths_mem, node_belief_states, node_probabilities,
           node_sequences, node_sequence_lengths, idx):
    idx2 = idx.reshape(B // CH, CH)
    idxw = idx.reshape(NW, PERW)
    nlen2 = node_sequence_lengths.reshape(NW, PERW)
    pad = jnp.zeros((LPADBIG,), jnp.int32)
    l_r = jax.new_ref(jnp.concatenate([sequence_lengths_mem, pad]))
    _sc_lens(l_r, nlen2, idx2)
    s_r = jax.new_ref(sequences_mem)
    _sc_scatter_rows(s_r, node_sequences, idxw)
    fout = _copy_call(belief_states_mem, probabilities_mem)
    n33 = _node_call(node_belief_states, node_probabilities)
    f_r = jax.new_ref(fout)
    _sc_scatter_rows(f_r, n33, idxw)
    return (f_r[...], s_r[...], l_r[...][:M], jnp.asarray(B, jnp.int32))
